# baseline (speedup 1.0000x reference)
import functools

import jax
import jax.numpy as jnp
from jax import lax
from jax.experimental import pallas as pl
from jax.experimental.pallas import tpu as pltpu
from jax.experimental.pallas import tpu_sc as plsc

_D = 512
_NC = 2
_NS = 16
_NW = _NC * _NS
_BN = 256
_BE = 256
_GC = 128
_SCW = 128


def _bdot(x, w):
    return jnp.dot(x.astype(jnp.bfloat16), w,
                   preferred_element_type=jnp.float32)


def _ln(x, g, b, eps=1e-5):
    mu = jnp.mean(x, axis=-1, keepdims=True)
    xc = x - mu
    var = jnp.mean(xc * xc, axis=-1, keepdims=True)
    return xc * jax.lax.rsqrt(var + eps) * g + b


def _node_mlp_body(h_ref, w1s, b1s, w2s, b2s, w1o, b1o, w2o, b2o, ps_o, po_o):
    hb = h_ref[...]
    t = jnp.maximum(_bdot(hb, w1s[...]) + b1s[...], 0.0)
    ps_o[...] = (_bdot(t, w2s[...]) + b2s[...]).astype(jnp.bfloat16)
    t = jnp.maximum(_bdot(hb, w1o[...]) + b1o[...], 0.0)
    po_o[...] = (_bdot(t, w2o[...]) + b2o[...]).astype(jnp.bfloat16)


def _node_mlp(h_p, w1s, b1s, w2s, b2s, w1o, b1o, w2o, b2o):
    n_pad = h_p.shape[0]
    blk = pl.BlockSpec((_BN, _D), lambda i: (i, 0))
    wsp = pl.BlockSpec((_D, _D), lambda i: (0, 0))
    bsp = pl.BlockSpec((1, _D), lambda i: (0, 0))
    return pl.pallas_call(
        _node_mlp_body,
        grid=(n_pad // _BN,),
        in_specs=[blk, wsp, bsp, wsp, bsp, wsp, bsp, wsp, bsp],
        out_specs=(blk, blk),
        out_shape=(jax.ShapeDtypeStruct((n_pad, _D), jnp.bfloat16),) * 2,
    )(h_p, w1s, b1s, w2s, b2s, w1o, b1o, w2o, b2o)


def _edge_mlp_body(x_ref, w1, b1, w2, b2, out_o):
    t = jnp.maximum(_bdot(x_ref[...], w1[...]) + b1[...], 0.0)
    out_o[...] = (_bdot(t, w2[...]) + b2[...]).astype(jnp.bfloat16)


def _edge_mlp(x, w1, b1, w2, b2):
    e_pad = x.shape[0]
    blk = pl.BlockSpec((_BE, _D), lambda i: (i, 0))
    wsp = pl.BlockSpec((_D, _D), lambda i: (0, 0))
    bsp = pl.BlockSpec((1, _D), lambda i: (0, 0))
    return pl.pallas_call(
        _edge_mlp_body,
        grid=(e_pad // _BE,),
        in_specs=[blk, wsp, bsp, wsp, bsp],
        out_specs=blk,
        out_shape=jax.ShapeDtypeStruct((e_pad, _D), jnp.bfloat16),
    )(x, w1, b1, w2, b2)


def _gru(x, h, wih, whh, bih, bhh):
    gi = _bdot(x, wih) + bih
    gh = _bdot(h, whh) + bhh
    r = jax.nn.sigmoid(gi[:, :_D] + gh[:, :_D])
    z = jax.nn.sigmoid(gi[:, _D:2 * _D] + gh[:, _D:2 * _D])
    n = jnp.tanh(gi[:, 2 * _D:] + r * gh[:, 2 * _D:])
    return (1.0 - z) * n + z * h


def _edge_mix_body(psg, pog, ppb, he, hes, wih, whh, bih, bhh, g, b,
                   ms_o, mo_o, he_o, hes_o):
    a = psg[...].astype(jnp.float32)
    o = pog[...].astype(jnp.float32)
    p = ppb[...].astype(jnp.float32)
    gv = g[...]
    bv = b[...]
    ms_o[...] = _ln(o + p, gv, bv)
    mo_o[...] = _ln(a + p, gv, bv)
    mp = _ln(a + o, gv, bv)
    hn = _gru(mp, he[...], wih[...], whh[...], bih[...], bhh[...])
    he_o[...] = hn
    hes_o[...] = hes[...] + hn


def _edge_mix(psg, pog, pp, he, hes, wih, whh, bih, bhh, g, b):
    e_pad = psg.shape[0]
    blk = pl.BlockSpec((_BE, _D), lambda i: (i, 0))
    wsp = pl.BlockSpec((_D, 3 * _D), lambda i: (0, 0))
    b3 = pl.BlockSpec((1, 3 * _D), lambda i: (0, 0))
    b1 = pl.BlockSpec((1, _D), lambda i: (0, 0))
    return pl.pallas_call(
        _edge_mix_body,
        grid=(e_pad // _BE,),
        in_specs=[blk, blk, blk, blk, blk, wsp, wsp, b3, b3, b1, b1],
        out_specs=(blk, blk, blk, blk),
        out_shape=(jax.ShapeDtypeStruct((e_pad, _D), jnp.float32),) * 4,
    )(psg, pog, pp, he, hes, wih, whh, bih, bhh, g, b)


def _node_upd_body(mpool, cnt, h, hs, wih, whh, bih, bhh, h_o, hs_o):
    c = jnp.maximum(cnt[...][:, :1], 1.0)
    mn = mpool[...] / c
    hn = _gru(mn, h[...], wih[...], whh[...], bih[...], bhh[...])
    h_o[...] = hn
    hs_o[...] = hs[...] + hn


def _node_upd(mpool, cnt, h, hs, wih, whh, bih, bhh):
    n_pad = mpool.shape[0]
    blk = pl.BlockSpec((_BN, _D), lambda i: (i, 0))
    cblk = pl.BlockSpec((_BN, 16), lambda i: (i, 0))
    wsp = pl.BlockSpec((_D, 3 * _D), lambda i: (0, 0))
    b3 = pl.BlockSpec((1, 3 * _D), lambda i: (0, 0))
    return pl.pallas_call(
        _node_upd_body,
        grid=(n_pad // _BN,),
        in_specs=[blk, cblk, blk, blk, wsp, wsp, b3, b3],
        out_specs=(blk, blk),
        out_shape=(jax.ShapeDtypeStruct((n_pad, _D), jnp.float32),) * 2,
    )(mpool, cnt, h, hs, wih, whh, bih, bhh)


def _final_ln_body(x, g, b, o):
    o[...] = _ln(x[...], g[...], b[...])


def _final_ln(x, g, b):
    rows = x.shape[0]
    blk = pl.BlockSpec((_BE, _D), lambda i: (i, 0))
    bsp = pl.BlockSpec((1, _D), lambda i: (0, 0))
    return pl.pallas_call(
        _final_ln_body,
        grid=(rows // _BE,),
        in_specs=[blk, bsp, bsp],
        out_specs=blk,
        out_shape=jax.ShapeDtypeStruct((rows, _D), jnp.float32),
    )(x, g, b)


def _make_gather(n_pad, e_pad):
    dw = _D // 2
    epw = e_pad // _NW
    gc = 128
    nk = epw // gc
    assert epw % gc == 0
    mesh = plsc.VectorSubcoreMesh(core_axis_name="c", subcore_axis_name="s",
                                  num_cores=_NC, num_subcores=_NS)

    def body(ps_hbm, po_hbm, sidx_hbm, oidx_hbm, psg_out, pog_out,
             idx_v, rows0, rows1, gsem0, gsem1, osem0, osem1):
        rows = (rows0, rows1)
        gsem = (gsem0, gsem1)
        osem = (osem0, osem1)
        c = lax.axis_index("c")
        s = lax.axis_index("s")
        wid = s * _NC + c
        base = wid * epw
        for tbl, idxh, outh in ((ps_hbm, sidx_hbm, psg_out),
                                (po_hbm, oidx_hbm, pog_out)):
            pltpu.sync_copy(idxh.at[pl.ds(base, epw)], idx_v)

            def gstart(k):
                b = k % 2
                return pltpu.async_copy(
                    tbl.at[idx_v.at[pl.ds(k * gc, gc)]], rows[b], gsem[b])

            def ostart(k):
                b = k % 2
                return pltpu.async_copy(
                    rows[b], outh.at[pl.ds(base + k * gc, gc), :], osem[b])

            gh = [None] * nk
            oh = [None] * nk
            gh[0] = gstart(0)
            for k in range(nk):
                gh[k].wait()
                if k + 1 < nk:
                    if k >= 1:
                        oh[k - 1].wait()
                    gh[k + 1] = gstart(k + 1)
                oh[k] = ostart(k)
            oh[nk - 1].wait()
            if nk >= 2:
                oh[nk - 2].wait()

    return pl.kernel(
        body,
        out_type=(jax.ShapeDtypeStruct((e_pad, dw), jnp.float32),) * 2,
        mesh=mesh,
        scratch_types=[
            pltpu.VMEM((epw,), jnp.int32),
            pltpu.VMEM((gc, dw), jnp.float32),
            pltpu.VMEM((gc, dw), jnp.float32),
            pltpu.SemaphoreType.DMA,
            pltpu.SemaphoreType.DMA,
            pltpu.SemaphoreType.DMA,
            pltpu.SemaphoreType.DMA,
        ],
    )


def _make_scatter(n_pad, e_pad, ncols, njc, writeback_core0_only):
    cw = min(_SCW, ncols)
    ept = e_pad // _NS
    npt = n_pad // _NS
    nke = ept // _GC
    nkn = npt // _GC
    assert ept % _GC == 0 and npt % _GC == 0
    mesh = plsc.VectorSubcoreMesh(core_axis_name="c", subcore_axis_name="s",
                                  num_cores=_NC, num_subcores=_NS)

    def body(ms_hbm, mo_hbm, sidx_hbm, oidx_hbm, zeros_hbm, out_hbm,
             idx_s, idx_o, val0, val1, acc_sp,
             vsem0, vsem1, asem0, asem1, osem):
        val = (val0, val1)
        vsem = (vsem0, vsem1)
        asem = (asem0, asem1)
        c = lax.axis_index("c")
        s = lax.axis_index("s")
        pltpu.sync_copy(sidx_hbm.at[pl.ds(s * nke, nke), :], idx_s)
        pltpu.sync_copy(oidx_hbm.at[pl.ds(s * nke, nke), :], idx_o)
        for j in range(njc):
            if writeback_core0_only:
                col = j * cw
            else:
                col = c * cw + j * (2 * cw)
            pltpu.sync_copy(zeros_hbm.at[pl.ds(0, _GC), pl.ds(0, cw)], val0)
            zh = [pltpu.async_copy(val0, acc_sp.at[pl.ds(s * npt + k * _GC,
                                                         _GC), :], vsem0)
                  for k in range(nkn)]
            for h in zh:
                h.wait()
            plsc.subcore_barrier()

            for arr, idx2 in ((ms_hbm, idx_s), (mo_hbm, idx_o)):
                def vstart(k):
                    b = k % 2
                    r0 = s * ept + k * _GC
                    return pltpu.async_copy(
                        arr.at[pl.ds(r0, _GC), pl.ds(col, cw)], val[b],
                        vsem[b])

                def astart(k):
                    b = k % 2
                    return pltpu.async_copy(val[b], acc_sp.at[idx2.at[k]],
                                            asem[b], add=True)

                vh = [None] * nke
                ah = [None] * nke
                vh[0] = vstart(0)
                for k in range(nke):
                    vh[k].wait()
                    if k + 1 < nke:
                        if k >= 1:
                            ah[k - 1].wait()
                        vh[k + 1] = vstart(k + 1)
                    ah[k] = astart(k)
                ah[nke - 1].wait()
                if nke >= 2:
                    ah[nke - 2].wait()
            plsc.subcore_barrier()

            def writeback():
                wh = [None] * nkn
                for k in range(nkn):
                    b = k % 2
                    if k >= 2:
                        wh[k - 2].wait()
                    r0 = s * npt + k * _GC
                    pltpu.sync_copy(acc_sp.at[pl.ds(r0, _GC), :], val[b])
                    wh[k] = pltpu.async_copy(
                        val[b], out_hbm.at[pl.ds(r0, _GC), pl.ds(col, cw)],
                        osem)
                for k in range(max(0, nkn - 2), nkn):
                    wh[k].wait()

            if writeback_core0_only:
                pl.when(c == 0)(writeback)
            else:
                writeback()
            if j + 1 < njc:
                plsc.subcore_barrier()

    return pl.kernel(
        body,
        out_type=jax.ShapeDtypeStruct((n_pad, ncols), jnp.float32),
        mesh=mesh,
        scratch_types=[
            pltpu.VMEM((nke, _GC), jnp.int32),
            pltpu.VMEM((nke, _GC), jnp.int32),
            pltpu.VMEM((_GC, cw), jnp.float32),
            pltpu.VMEM((_GC, cw), jnp.float32),
            pltpu.VMEM_SHARED((n_pad, cw), jnp.float32),
            pltpu.SemaphoreType.DMA,
            pltpu.SemaphoreType.DMA,
            pltpu.SemaphoreType.DMA,
            pltpu.SemaphoreType.DMA,
            pltpu.SemaphoreType.DMA,
        ],
    )


def kernel(h, h_edge, params, edge_index):
    n, d = h.shape
    e = h_edge.shape[0]
    num_layers = params["phis_W1"].shape[0]
    n_pad = ((n + _BN - 1) // _BN) * _BN
    if n_pad % (_NS * _GC) != 0:
        n_pad = ((n + _NS * _GC - 1) // (_NS * _GC)) * (_NS * _GC)
    e_pad = ((e + _NW * _GC - 1) // (_NW * _GC)) * (_NW * _GC)

    trash = n_pad - 1
    h_p = jnp.pad(h, ((0, n_pad - n), (0, 0)))
    he_p = jnp.pad(h_edge, ((0, e_pad - e), (0, 0)))
    sidx = jnp.pad(edge_index[0], (0, e_pad - e), constant_values=trash)
    oidx = jnp.pad(edge_index[1], (0, e_pad - e), constant_values=trash)
    zeros_blk = jnp.zeros((_GC, _SCW), jnp.float32)
    ones_e = jnp.ones((e_pad, _SCW), jnp.float32)

    gather = _make_gather(n_pad, e_pad)
    scatter = _make_scatter(n_pad, e_pad, _D, _D // (2 * _SCW), False)
    count_k = _make_scatter(n_pad, e_pad, _SCW, 1, True)

    sidx2 = sidx.reshape(e_pad // _GC, _GC)
    oidx2 = oidx.reshape(e_pad // _GC, _GC)

    counts = count_k(ones_e, ones_e, sidx2, oidx2, zeros_blk)[:, :16]

    def wT(x):
        return jnp.swapaxes(x, 0, 1).astype(jnp.bfloat16)

    def row(x):
        return x.reshape(1, -1)

    h_sum = jnp.zeros((n_pad, d), jnp.float32)
    he_sum = jnp.zeros((e_pad, d), jnp.float32)

    for i in range(num_layers):
        ps_all, po_all = _node_mlp(
            h_p,
            wT(params["phis_W1"][i]), row(params["phis_b1"][i]),
            wT(params["phis_W2"][i]), row(params["phis_b2"][i]),
            wT(params["phio_W1"][i]), row(params["phio_b1"][i]),
            wT(params["phio_W2"][i]), row(params["phio_b2"][i]))
        pp = _edge_mlp(
            he_p,
            wT(params["phip_W1"][i]), row(params["phip_b1"][i]),
            wT(params["phip_W2"][i]), row(params["phip_b2"][i]))
        ps_v = lax.bitcast_convert_type(
            ps_all.reshape(n_pad, _D // 2, 2), jnp.float32)
        po_v = lax.bitcast_convert_type(
            po_all.reshape(n_pad, _D // 2, 2), jnp.float32)
        psg_v, pog_v = gather(ps_v, po_v, sidx, oidx)
        psg = lax.bitcast_convert_type(psg_v, jnp.bfloat16).reshape(e_pad, _D)
        pog = lax.bitcast_convert_type(pog_v, jnp.bfloat16).reshape(e_pad, _D)
        ms, mo, he_p, he_sum = _edge_mix(
            psg, pog, pp, he_p, he_sum,
            wT(params["edge_gru_Wih"][i]), wT(params["edge_gru_Whh"][i]),
            row(params["edge_gru_bih"][i]), row(params["edge_gru_bhh"][i]),
            row(params["ln_g"][i]), row(params["ln_b"][i]))
        mpool = scatter(ms, mo, sidx2, oidx2, zeros_blk)
        h_p, h_sum = _node_upd(
            mpool, counts, h_p, h_sum,
            wT(params["node_gru_Wih"][i]), wT(params["node_gru_Whh"][i]),
            row(params["node_gru_bih"][i]), row(params["node_gru_bhh"][i]))

    g = row(params["final_ln_g"])
    b = row(params["final_ln_b"])
    h_final = _final_ln(h_sum, g, b)[:n]
    he_final = _final_ln(he_sum, g, b)[:e]
    return (h_final, he_final)


# speedup vs baseline: 1.9067x; 1.9067x over previous
import functools

import jax
import jax.numpy as jnp
from jax import lax
from jax.experimental import pallas as pl
from jax.experimental.pallas import tpu as pltpu
from jax.experimental.pallas import tpu_sc as plsc

_D = 512
_NC = 2
_NS = 16
_NW = _NC * _NS
_BN = 256
_BE = 256
_GC = 128
_SCW = 128


def _bdot(x, w):
    return jnp.dot(x.astype(jnp.bfloat16), w,
                   preferred_element_type=jnp.float32)


def _pack2(u, v):
    ui = lax.bitcast_convert_type(u.astype(jnp.bfloat16).astype(jnp.float32),
                                  jnp.uint32)
    vi = lax.bitcast_convert_type(v.astype(jnp.bfloat16).astype(jnp.float32),
                                  jnp.uint32)
    w = (ui >> 16) | (vi & jnp.uint32(0xFFFF0000))
    return lax.bitcast_convert_type(w, jnp.float32)


def _unpack2(w):
    wi = lax.bitcast_convert_type(w, jnp.uint32)
    u = lax.bitcast_convert_type(wi << 16, jnp.float32)
    v = lax.bitcast_convert_type(wi & jnp.uint32(0xFFFF0000), jnp.float32)
    return u, v


def _unpack_cat(w):
    u, v = _unpack2(w)
    return jnp.concatenate([u, v], axis=1)


def _ln(x, g, b, eps=1e-5):
    mu = jnp.mean(x, axis=-1, keepdims=True)
    xc = x - mu
    var = jnp.mean(xc * xc, axis=-1, keepdims=True)
    return xc * jax.lax.rsqrt(var + eps) * g + b


def _node_mlp_body(h_ref, w1s, b1s, w2s, b2s, w1o, b1o, w2o, b2o, ps_o, po_o):
    hb = h_ref[...]
    hw = _D // 2
    t = jnp.maximum(_bdot(hb, w1s[...]) + b1s[...], 0.0)
    z = _bdot(t, w2s[...]) + b2s[...]
    ps_o[...] = _pack2(z[:, :hw], z[:, hw:])
    t = jnp.maximum(_bdot(hb, w1o[...]) + b1o[...], 0.0)
    z = _bdot(t, w2o[...]) + b2o[...]
    po_o[...] = _pack2(z[:, :hw], z[:, hw:])


def _node_mlp(h_p, w1s, b1s, w2s, b2s, w1o, b1o, w2o, b2o):
    n_pad = h_p.shape[0]
    blk = pl.BlockSpec((_BN, _D), lambda i: (i, 0))
    pblk = pl.BlockSpec((_BN, _D // 2), lambda i: (i, 0))
    wsp = pl.BlockSpec((_D, _D), lambda i: (0, 0))
    bsp = pl.BlockSpec((1, _D), lambda i: (0, 0))
    return pl.pallas_call(
        _node_mlp_body,
        grid=(n_pad // _BN,),
        in_specs=[blk, wsp, bsp, wsp, bsp, wsp, bsp, wsp, bsp],
        out_specs=(pblk, pblk),
        out_shape=(jax.ShapeDtypeStruct((n_pad, _D // 2), jnp.float32),) * 2,
    )(h_p, w1s, b1s, w2s, b2s, w1o, b1o, w2o, b2o)


def _edge_mlp_body(x_ref, w1, b1, w2, b2, out_o):
    t = jnp.maximum(_bdot(x_ref[...], w1[...]) + b1[...], 0.0)
    z = _bdot(t, w2[...]) + b2[...]
    out_o[...] = _pack2(z[:, :_D // 2], z[:, _D // 2:])


def _edge_mlp(x, w1, b1, w2, b2):
    e_pad = x.shape[0]
    blk = pl.BlockSpec((_BE, _D), lambda i: (i, 0))
    pblk = pl.BlockSpec((_BE, _D // 2), lambda i: (i, 0))
    wsp = pl.BlockSpec((_D, _D), lambda i: (0, 0))
    bsp = pl.BlockSpec((1, _D), lambda i: (0, 0))
    return pl.pallas_call(
        _edge_mlp_body,
        grid=(e_pad // _BE,),
        in_specs=[blk, wsp, bsp, wsp, bsp],
        out_specs=pblk,
        out_shape=jax.ShapeDtypeStruct((e_pad, _D // 2), jnp.float32),
    )(x, w1, b1, w2, b2)


def _gru(x, h, wih, whh, bih, bhh):
    gi = _bdot(x, wih) + bih
    gh = _bdot(h, whh) + bhh
    r = jax.nn.sigmoid(gi[:, :_D] + gh[:, :_D])
    z = jax.nn.sigmoid(gi[:, _D:2 * _D] + gh[:, _D:2 * _D])
    n = jnp.tanh(gi[:, 2 * _D:] + r * gh[:, 2 * _D:])
    return (1.0 - z) * n + z * h


def _edge_mix_body(psg, pog, ppb, he, hes, wih, whh, bih, bhh, g, b,
                   ms_o, mo_o, he_o, hes_o):
    a = _unpack_cat(psg[...])
    o = _unpack_cat(pog[...])
    p = _unpack_cat(ppb[...])
    gv = g[...]
    bv = b[...]
    ms_o[...] = _ln(o + p, gv, bv)
    mo_o[...] = _ln(a + p, gv, bv)
    mp = _ln(a + o, gv, bv)
    hn = _gru(mp, he[...], wih[...], whh[...], bih[...], bhh[...])
    he_o[...] = hn
    hes_o[...] = hes[...] + hn


def _edge_mix(psg, pog, pp, he, hes, wih, whh, bih, bhh, g, b):
    e_pad = psg.shape[0]
    blk = pl.BlockSpec((_BE, _D), lambda i: (i, 0))
    pblk = pl.BlockSpec((_BE, _D // 2), lambda i: (i, 0))
    wsp = pl.BlockSpec((_D, 3 * _D), lambda i: (0, 0))
    b3 = pl.BlockSpec((1, 3 * _D), lambda i: (0, 0))
    b1 = pl.BlockSpec((1, _D), lambda i: (0, 0))
    return pl.pallas_call(
        _edge_mix_body,
        grid=(e_pad // _BE,),
        in_specs=[pblk, pblk, pblk, blk, blk, wsp, wsp, b3, b3, b1, b1],
        out_specs=(blk, blk, blk, blk),
        out_shape=(jax.ShapeDtypeStruct((e_pad, _D), jnp.float32),) * 4,
    )(psg, pog, pp, he, hes, wih, whh, bih, bhh, g, b)


def _node_upd_body(mpool, cnt, h, hs, wih, whh, bih, bhh, h_o, hs_o):
    c = jnp.maximum(cnt[...][:, :1], 1.0)
    mn = mpool[...] / c
    hn = _gru(mn, h[...], wih[...], whh[...], bih[...], bhh[...])
    h_o[...] = hn
    hs_o[...] = hs[...] + hn


def _node_upd(mpool, cnt, h, hs, wih, whh, bih, bhh):
    n_pad = mpool.shape[0]
    blk = pl.BlockSpec((_BN, _D), lambda i: (i, 0))
    cblk = pl.BlockSpec((_BN, 16), lambda i: (i, 0))
    wsp = pl.BlockSpec((_D, 3 * _D), lambda i: (0, 0))
    b3 = pl.BlockSpec((1, 3 * _D), lambda i: (0, 0))
    return pl.pallas_call(
        _node_upd_body,
        grid=(n_pad // _BN,),
        in_specs=[blk, cblk, blk, blk, wsp, wsp, b3, b3],
        out_specs=(blk, blk),
        out_shape=(jax.ShapeDtypeStruct((n_pad, _D), jnp.float32),) * 2,
    )(mpool, cnt, h, hs, wih, whh, bih, bhh)


def _final_ln_body(x, g, b, o):
    o[...] = _ln(x[...], g[...], b[...])


def _final_ln(x, g, b):
    rows = x.shape[0]
    blk = pl.BlockSpec((_BE, _D), lambda i: (i, 0))
    bsp = pl.BlockSpec((1, _D), lambda i: (0, 0))
    return pl.pallas_call(
        _final_ln_body,
        grid=(rows // _BE,),
        in_specs=[blk, bsp, bsp],
        out_specs=blk,
        out_shape=jax.ShapeDtypeStruct((rows, _D), jnp.float32),
    )(x, g, b)


def _make_gather(n_pad, e_pad):
    dw = _D // 2
    epw = e_pad // _NW
    gc = 128
    nk = epw // gc
    assert epw % gc == 0
    mesh = plsc.VectorSubcoreMesh(core_axis_name="c", subcore_axis_name="s",
                                  num_cores=_NC, num_subcores=_NS)

    def body(ps_hbm, po_hbm, sidx_hbm, oidx_hbm, psg_out, pog_out,
             idx_v, rows0, rows1, gsem0, gsem1, osem0, osem1):
        rows = (rows0, rows1)
        gsem = (gsem0, gsem1)
        osem = (osem0, osem1)
        c = lax.axis_index("c")
        s = lax.axis_index("s")
        wid = s * _NC + c
        base = wid * epw
        for tbl, idxh, outh in ((ps_hbm, sidx_hbm, psg_out),
                                (po_hbm, oidx_hbm, pog_out)):
            pltpu.sync_copy(idxh.at[pl.ds(base, epw)], idx_v)

            def gstart(k):
                b = k % 2
                return pltpu.async_copy(
                    tbl.at[idx_v.at[pl.ds(k * gc, gc)]], rows[b], gsem[b])

            def ostart(k):
                b = k % 2
                return pltpu.async_copy(
                    rows[b], outh.at[pl.ds(base + k * gc, gc), :], osem[b])

            gh = [None] * nk
            oh = [None] * nk
            gh[0] = gstart(0)
            for k in range(nk):
                gh[k].wait()
                if k + 1 < nk:
                    if k >= 1:
                        oh[k - 1].wait()
                    gh[k + 1] = gstart(k + 1)
                oh[k] = ostart(k)
            oh[nk - 1].wait()
            if nk >= 2:
                oh[nk - 2].wait()

    return pl.kernel(
        body,
        out_type=(jax.ShapeDtypeStruct((e_pad, dw), jnp.float32),) * 2,
        mesh=mesh,
        scratch_types=[
            pltpu.VMEM((epw,), jnp.int32),
            pltpu.VMEM((gc, dw), jnp.float32),
            pltpu.VMEM((gc, dw), jnp.float32),
            pltpu.SemaphoreType.DMA,
            pltpu.SemaphoreType.DMA,
            pltpu.SemaphoreType.DMA,
            pltpu.SemaphoreType.DMA,
        ],
    )


def _make_scatter(n_pad, e_pad, ncols, njc, writeback_core0_only):
    cw = min(_SCW, ncols)
    ept = e_pad // _NS
    npt = n_pad // _NS
    nke = ept // _GC
    nkn = npt // _GC
    assert ept % _GC == 0 and npt % _GC == 0
    mesh = plsc.VectorSubcoreMesh(core_axis_name="c", subcore_axis_name="s",
                                  num_cores=_NC, num_subcores=_NS)

    def body(ms_hbm, mo_hbm, sidx_hbm, oidx_hbm, zeros_hbm, out_hbm,
             idx_s, idx_o, val0, val1, acc_sp,
             vsem0, vsem1, asem0, asem1, osem):
        val = (val0, val1)
        vsem = (vsem0, vsem1)
        asem = (asem0, asem1)
        c = lax.axis_index("c")
        s = lax.axis_index("s")
        pltpu.sync_copy(sidx_hbm.at[pl.ds(s * nke, nke), :], idx_s)
        pltpu.sync_copy(oidx_hbm.at[pl.ds(s * nke, nke), :], idx_o)
        for j in range(njc):
            if writeback_core0_only:
                col = j * cw
            else:
                col = c * cw + j * (2 * cw)
            pltpu.sync_copy(zeros_hbm.at[pl.ds(0, _GC), pl.ds(0, cw)], val0)
            zh = [pltpu.async_copy(val0, acc_sp.at[pl.ds(s * npt + k * _GC,
                                                         _GC), :], vsem0)
                  for k in range(nkn)]
            for h in zh:
                h.wait()
            plsc.subcore_barrier()

            for arr, idx2 in ((ms_hbm, idx_s), (mo_hbm, idx_o)):
                def vstart(k):
                    b = k % 2
                    r0 = s * ept + k * _GC
                    return pltpu.async_copy(
                        arr.at[pl.ds(r0, _GC), pl.ds(col, cw)], val[b],
                        vsem[b])

                def astart(k):
                    b = k % 2
                    return pltpu.async_copy(val[b], acc_sp.at[idx2.at[k]],
                                            asem[b], add=True)

                vh = [None] * nke
                ah = [None] * nke
                vh[0] = vstart(0)
                for k in range(nke):
                    vh[k].wait()
                    if k + 1 < nke:
                        if k >= 1:
                            ah[k - 1].wait()
                        vh[k + 1] = vstart(k + 1)
                    ah[k] = astart(k)
                ah[nke - 1].wait()
                if nke >= 2:
                    ah[nke - 2].wait()
            plsc.subcore_barrier()

            def writeback():
                wh = [None] * nkn
                for k in range(nkn):
                    b = k % 2
                    if k >= 2:
                        wh[k - 2].wait()
                    r0 = s * npt + k * _GC
                    pltpu.sync_copy(acc_sp.at[pl.ds(r0, _GC), :], val[b])
                    wh[k] = pltpu.async_copy(
                        val[b], out_hbm.at[pl.ds(r0, _GC), pl.ds(col, cw)],
                        osem)
                for k in range(max(0, nkn - 2), nkn):
                    wh[k].wait()

            if writeback_core0_only:
                pl.when(c == 0)(writeback)
            else:
                writeback()
            if j + 1 < njc:
                plsc.subcore_barrier()

    return pl.kernel(
        body,
        out_type=jax.ShapeDtypeStruct((n_pad, ncols), jnp.float32),
        mesh=mesh,
        scratch_types=[
            pltpu.VMEM((nke, _GC), jnp.int32),
            pltpu.VMEM((nke, _GC), jnp.int32),
            pltpu.VMEM((_GC, cw), jnp.float32),
            pltpu.VMEM((_GC, cw), jnp.float32),
            pltpu.VMEM_SHARED((n_pad, cw), jnp.float32),
            pltpu.SemaphoreType.DMA,
            pltpu.SemaphoreType.DMA,
            pltpu.SemaphoreType.DMA,
            pltpu.SemaphoreType.DMA,
            pltpu.SemaphoreType.DMA,
        ],
    )


def kernel(h, h_edge, params, edge_index):
    n, d = h.shape
    e = h_edge.shape[0]
    num_layers = params["phis_W1"].shape[0]
    n_pad = ((n + _BN - 1) // _BN) * _BN
    if n_pad % (_NS * _GC) != 0:
        n_pad = ((n + _NS * _GC - 1) // (_NS * _GC)) * (_NS * _GC)
    e_pad = ((e + _NW * _GC - 1) // (_NW * _GC)) * (_NW * _GC)

    trash = n_pad - 1
    h_p = jnp.pad(h, ((0, n_pad - n), (0, 0)))
    he_p = jnp.pad(h_edge, ((0, e_pad - e), (0, 0)))
    sidx = jnp.pad(edge_index[0], (0, e_pad - e), constant_values=trash)
    oidx = jnp.pad(edge_index[1], (0, e_pad - e), constant_values=trash)
    zeros_blk = jnp.zeros((_GC, _SCW), jnp.float32)
    ones_e = jnp.ones((e_pad, _SCW), jnp.float32)

    gather = _make_gather(n_pad, e_pad)
    scatter = _make_scatter(n_pad, e_pad, _D, _D // (2 * _SCW), False)
    count_k = _make_scatter(n_pad, e_pad, _SCW, 1, True)

    sidx2 = sidx.reshape(e_pad // _GC, _GC)
    oidx2 = oidx.reshape(e_pad // _GC, _GC)

    counts = count_k(ones_e, ones_e, sidx2, oidx2, zeros_blk)[:, :16]

    def wT(x):
        return jnp.swapaxes(x, 0, 1).astype(jnp.bfloat16)

    def row(x):
        return x.reshape(1, -1)

    h_sum = jnp.zeros((n_pad, d), jnp.float32)
    he_sum = jnp.zeros((e_pad, d), jnp.float32)

    for i in range(num_layers):
        ps_all, po_all = _node_mlp(
            h_p,
            wT(params["phis_W1"][i]), row(params["phis_b1"][i]),
            wT(params["phis_W2"][i]), row(params["phis_b2"][i]),
            wT(params["phio_W1"][i]), row(params["phio_b1"][i]),
            wT(params["phio_W2"][i]), row(params["phio_b2"][i]))
        pp = _edge_mlp(
            he_p,
            wT(params["phip_W1"][i]), row(params["phip_b1"][i]),
            wT(params["phip_W2"][i]), row(params["phip_b2"][i]))
        psg, pog = gather(ps_all, po_all, sidx, oidx)
        ms, mo, he_p, he_sum = _edge_mix(
            psg, pog, pp, he_p, he_sum,
            wT(params["edge_gru_Wih"][i]), wT(params["edge_gru_Whh"][i]),
            row(params["edge_gru_bih"][i]), row(params["edge_gru_bhh"][i]),
            row(params["ln_g"][i]), row(params["ln_b"][i]))
        mpool = scatter(ms, mo, sidx2, oidx2, zeros_blk)
        h_p, h_sum = _node_upd(
            mpool, counts, h_p, h_sum,
            wT(params["node_gru_Wih"][i]), wT(params["node_gru_Whh"][i]),
            row(params["node_gru_bih"][i]), row(params["node_gru_bhh"][i]))

    g = row(params["final_ln_g"])
    b = row(params["final_ln_b"])
    h_final = _final_ln(h_sum, g, b)[:n]
    he_final = _final_ln(he_sum, g, b)[:e]
    return (h_final, he_final)


# speedup vs baseline: 2.1440x; 1.1245x over previous
import functools

import jax
import jax.numpy as jnp
from jax import lax
from jax.experimental import pallas as pl
from jax.experimental.pallas import tpu as pltpu
from jax.experimental.pallas import tpu_sc as plsc

_D = 512
_NC = 2
_NS = 16
_NW = _NC * _NS
_BN = 256
_BE = 256
_GC = 128
_SCW = 128
_EC = 64


def _bdot(x, w):
    return jnp.dot(x.astype(jnp.bfloat16), w,
                   preferred_element_type=jnp.float32)


def _pack2(u, v):
    ui = lax.bitcast_convert_type(u.astype(jnp.bfloat16).astype(jnp.float32),
                                  jnp.uint32)
    vi = lax.bitcast_convert_type(v.astype(jnp.bfloat16).astype(jnp.float32),
                                  jnp.uint32)
    w = (ui >> 16) | (vi & jnp.uint32(0xFFFF0000))
    return lax.bitcast_convert_type(w, jnp.float32)


def _unpack2(w):
    wi = lax.bitcast_convert_type(w, jnp.uint32)
    u = lax.bitcast_convert_type(wi << 16, jnp.float32)
    v = lax.bitcast_convert_type(wi & jnp.uint32(0xFFFF0000), jnp.float32)
    return u, v


def _unpack_cat(w):
    u, v = _unpack2(w)
    return jnp.concatenate([u, v], axis=1)


def _ln(x, g, b, eps=1e-5):
    mu = jnp.mean(x, axis=-1, keepdims=True)
    xc = x - mu
    var = jnp.mean(xc * xc, axis=-1, keepdims=True)
    return xc * jax.lax.rsqrt(var + eps) * g + b


def _node_mlp_body(h_ref, w1s, b1s, w2s, b2s, w1o, b1o, w2o, b2o, ps_o, po_o):
    hb = h_ref[...]
    hw = _D // 2
    t = jnp.maximum(_bdot(hb, w1s[...]) + b1s[...], 0.0)
    z = _bdot(t, w2s[...]) + b2s[...]
    ps_o[...] = _pack2(z[:, :hw], z[:, hw:])
    t = jnp.maximum(_bdot(hb, w1o[...]) + b1o[...], 0.0)
    z = _bdot(t, w2o[...]) + b2o[...]
    po_o[...] = _pack2(z[:, :hw], z[:, hw:])


def _node_mlp(h_p, w1s, b1s, w2s, b2s, w1o, b1o, w2o, b2o):
    n_pad = h_p.shape[0]
    blk = pl.BlockSpec((_BN, _D), lambda i: (i, 0))
    pblk = pl.BlockSpec((_BN, _D // 2), lambda i: (i, 0))
    wsp = pl.BlockSpec((_D, _D), lambda i: (0, 0))
    bsp = pl.BlockSpec((1, _D), lambda i: (0, 0))
    return pl.pallas_call(
        _node_mlp_body,
        grid=(n_pad // _BN,),
        in_specs=[blk, wsp, bsp, wsp, bsp, wsp, bsp, wsp, bsp],
        out_specs=(pblk, pblk),
        out_shape=(jax.ShapeDtypeStruct((n_pad, _D // 2), jnp.float32),) * 2,
    )(h_p, w1s, b1s, w2s, b2s, w1o, b1o, w2o, b2o)


def _edge_mlp_body(x_ref, w1, b1, w2, b2, out_o):
    t = jnp.maximum(_bdot(x_ref[...], w1[...]) + b1[...], 0.0)
    z = _bdot(t, w2[...]) + b2[...]
    out_o[...] = _pack2(z[:, :_D // 2], z[:, _D // 2:])


def _edge_mlp(x, w1, b1, w2, b2):
    e_pad = x.shape[0]
    blk = pl.BlockSpec((_BE, _D), lambda i: (i, 0))
    pblk = pl.BlockSpec((_BE, _D // 2), lambda i: (i, 0))
    wsp = pl.BlockSpec((_D, _D), lambda i: (0, 0))
    bsp = pl.BlockSpec((1, _D), lambda i: (0, 0))
    return pl.pallas_call(
        _edge_mlp_body,
        grid=(e_pad // _BE,),
        in_specs=[blk, wsp, bsp, wsp, bsp],
        out_specs=pblk,
        out_shape=jax.ShapeDtypeStruct((e_pad, _D // 2), jnp.float32),
    )(x, w1, b1, w2, b2)


def _gru(x, h, wih, whh, bih, bhh):
    gi = _bdot(x, wih) + bih
    gh = _bdot(h, whh) + bhh
    r = jax.nn.sigmoid(gi[:, :_D] + gh[:, :_D])
    z = jax.nn.sigmoid(gi[:, _D:2 * _D] + gh[:, _D:2 * _D])
    n = jnp.tanh(gi[:, 2 * _D:] + r * gh[:, 2 * _D:])
    return (1.0 - z) * n + z * h


def _edge_mix_body(psg, pog, ppb, he, hes, wih, whh, bih, bhh, g, b,
                   ms_o, mo_o, he_o, hes_o):
    a = _unpack_cat(psg[...])
    o = _unpack_cat(pog[...])
    p = _unpack_cat(ppb[...])
    gv = g[...]
    bv = b[...]
    ms_o[...] = _ln(o + p, gv, bv)
    mo_o[...] = _ln(a + p, gv, bv)
    mp = _ln(a + o, gv, bv)
    hn = _gru(mp, he[...], wih[...], whh[...], bih[...], bhh[...])
    he_o[...] = hn
    hes_o[...] = hes[...] + hn


def _edge_mix(psg, pog, pp, he, hes, wih, whh, bih, bhh, g, b):
    e_pad = psg.shape[0]
    blk = pl.BlockSpec((_BE, _D), lambda i: (i, 0))
    pblk = pl.BlockSpec((_BE, _D // 2), lambda i: (i, 0))
    wsp = pl.BlockSpec((_D, 3 * _D), lambda i: (0, 0))
    b3 = pl.BlockSpec((1, 3 * _D), lambda i: (0, 0))
    b1 = pl.BlockSpec((1, _D), lambda i: (0, 0))
    return pl.pallas_call(
        _edge_mix_body,
        grid=(e_pad // _BE,),
        in_specs=[pblk, pblk, pblk, blk, blk, wsp, wsp, b3, b3, b1, b1],
        out_specs=(blk, blk, blk, blk),
        out_shape=(jax.ShapeDtypeStruct((e_pad, _D), jnp.float32),) * 4,
    )(psg, pog, pp, he, hes, wih, whh, bih, bhh, g, b)


def _node_upd_body(mpool, cnt, h, hs, wih, whh, bih, bhh, h_o, hs_o):
    c = jnp.maximum(cnt[...][:, :1], 1.0)
    mn = mpool[...] / c
    hn = _gru(mn, h[...], wih[...], whh[...], bih[...], bhh[...])
    h_o[...] = hn
    hs_o[...] = hs[...] + hn


def _node_upd(mpool, cnt, h, hs, wih, whh, bih, bhh):
    n_pad = mpool.shape[0]
    blk = pl.BlockSpec((_BN, _D), lambda i: (i, 0))
    cblk = pl.BlockSpec((_BN, 16), lambda i: (i, 0))
    wsp = pl.BlockSpec((_D, 3 * _D), lambda i: (0, 0))
    b3 = pl.BlockSpec((1, 3 * _D), lambda i: (0, 0))
    return pl.pallas_call(
        _node_upd_body,
        grid=(n_pad // _BN,),
        in_specs=[blk, cblk, blk, blk, wsp, wsp, b3, b3],
        out_specs=(blk, blk),
        out_shape=(jax.ShapeDtypeStruct((n_pad, _D), jnp.float32),) * 2,
    )(mpool, cnt, h, hs, wih, whh, bih, bhh)


def _final_ln_body(x, g, b, o):
    o[...] = _ln(x[...], g[...], b[...])


def _final_ln(x, g, b):
    rows = x.shape[0]
    blk = pl.BlockSpec((_BE, _D), lambda i: (i, 0))
    bsp = pl.BlockSpec((1, _D), lambda i: (0, 0))
    return pl.pallas_call(
        _final_ln_body,
        grid=(rows // _BE,),
        in_specs=[blk, bsp, bsp],
        out_specs=blk,
        out_shape=jax.ShapeDtypeStruct((rows, _D), jnp.float32),
    )(x, g, b)


def _make_gather(n_pad, e_pad):
    dw = _D // 2
    epw = e_pad // _NW
    gc = 64
    nk = epw // gc
    assert epw % gc == 0
    mesh = plsc.VectorSubcoreMesh(core_axis_name="c", subcore_axis_name="s",
                                  num_cores=_NC, num_subcores=_NS)

    def body(ps_hbm, po_hbm, sidx_hbm, oidx_hbm, psg_out, pog_out,
             idx_v, r00, r01, r10, r11,
             gs00, gs01, gs10, gs11, os00, os01, os10, os11):
        rows = ((r00, r01), (r10, r11))
        gsem = ((gs00, gs01), (gs10, gs11))
        osem = ((os00, os01), (os10, os11))
        c = lax.axis_index("c")
        s = lax.axis_index("s")
        wid = s * _NC + c
        base = wid * epw
        tabs = ((ps_hbm, sidx_hbm, psg_out), (po_hbm, oidx_hbm, pog_out))
        pltpu.sync_copy(tabs[0][1].at[pl.ds(base, epw)], idx_v.at[0])
        pltpu.sync_copy(tabs[1][1].at[pl.ds(base, epw)], idx_v.at[1])

        def gstart(t, k):
            b = k % 2
            return pltpu.async_copy(
                tabs[t][0].at[idx_v.at[t, pl.ds(k * gc, gc)]],
                rows[t][b], gsem[t][b])

        def ostart(t, k):
            b = k % 2
            return pltpu.async_copy(
                rows[t][b], tabs[t][2].at[pl.ds(base + k * gc, gc), :],
                osem[t][b])

        gh = [[None] * nk, [None] * nk]
        oh = [[None] * nk, [None] * nk]
        gh[0][0] = gstart(0, 0)
        gh[1][0] = gstart(1, 0)
        for k in range(nk):
            for t in (0, 1):
                gh[t][k].wait()
                if k + 1 < nk:
                    if k >= 1:
                        oh[t][k - 1].wait()
                    gh[t][k + 1] = gstart(t, k + 1)
                oh[t][k] = ostart(t, k)
        for t in (0, 1):
            oh[t][nk - 1].wait()
            if nk >= 2:
                oh[t][nk - 2].wait()

    return pl.kernel(
        body,
        out_type=(jax.ShapeDtypeStruct((e_pad, dw), jnp.float32),) * 2,
        mesh=mesh,
        scratch_types=(
            [pltpu.VMEM((2, epw), jnp.int32)]
            + [pltpu.VMEM((gc, dw), jnp.float32)] * 4
            + [pltpu.SemaphoreType.DMA] * 8
        ),
    )


def _make_scatter(n_pad, e_pad, ncols, njc, writeback_core0_only):
    cw = min(_SCW, ncols)
    ept = e_pad // _NS
    npt = n_pad // _NS
    nke = ept // _EC
    nkn = npt // _EC
    assert ept % _EC == 0 and npt % _EC == 0
    mesh = plsc.VectorSubcoreMesh(core_axis_name="c", subcore_axis_name="s",
                                  num_cores=_NC, num_subcores=_NS)

    def body(ms_hbm, mo_hbm, sidx_hbm, oidx_hbm, zeros_hbm, out_hbm,
             idx_s, idx_o, v00, v01, v10, v11, acc_sp,
             vs00, vs01, vs10, vs11, as00, as01, as10, as11, osem):
        val = ((v00, v01), (v10, v11))
        vsem = ((vs00, vs01), (vs10, vs11))
        asem = ((as00, as01), (as10, as11))
        c = lax.axis_index("c")
        s = lax.axis_index("s")
        pltpu.sync_copy(sidx_hbm.at[pl.ds(s * nke, nke), :], idx_s)
        pltpu.sync_copy(oidx_hbm.at[pl.ds(s * nke, nke), :], idx_o)
        for j in range(njc):
            if writeback_core0_only:
                col = j * cw
            else:
                col = c * cw + j * (2 * cw)
            pltpu.sync_copy(zeros_hbm.at[pl.ds(0, _EC), pl.ds(0, cw)], v00)
            zh = [pltpu.async_copy(v00, acc_sp.at[pl.ds(s * npt + k * _EC,
                                                        _EC), :], vs00)
                  for k in range(nkn)]
            for h in zh:
                h.wait()
            plsc.subcore_barrier()

            arrs = ((ms_hbm, idx_s), (mo_hbm, idx_o))

            def vstart(t, k):
                b = k % 2
                r0 = s * ept + k * _EC
                return pltpu.async_copy(
                    arrs[t][0].at[pl.ds(r0, _EC), pl.ds(col, cw)],
                    val[t][b], vsem[t][b])

            def astart(t, k):
                b = k % 2
                return pltpu.async_copy(val[t][b],
                                        acc_sp.at[arrs[t][1].at[k]],
                                        asem[t][b], add=True)

            vh = [[None] * nke, [None] * nke]
            ah = [[None] * nke, [None] * nke]
            vh[0][0] = vstart(0, 0)
            vh[1][0] = vstart(1, 0)
            for k in range(nke):
                for t in (0, 1):
                    vh[t][k].wait()
                    if k + 1 < nke:
                        if k >= 1:
                            ah[t][k - 1].wait()
                        vh[t][k + 1] = vstart(t, k + 1)
                    ah[t][k] = astart(t, k)
            for t in (0, 1):
                ah[t][nke - 1].wait()
                if nke >= 2:
                    ah[t][nke - 2].wait()
            plsc.subcore_barrier()

            def writeback():
                wh = [None] * nkn
                for k in range(nkn):
                    b = k % 2
                    if k >= 2:
                        wh[k - 2].wait()
                    r0 = s * npt + k * _EC
                    pltpu.sync_copy(acc_sp.at[pl.ds(r0, _EC), :], val[0][b])
                    wh[k] = pltpu.async_copy(
                        val[0][b], out_hbm.at[pl.ds(r0, _EC), pl.ds(col, cw)],
                        osem)
                for k in range(max(0, nkn - 2), nkn):
                    wh[k].wait()

            if writeback_core0_only:
                pl.when(c == 0)(writeback)
            else:
                writeback()
            if j + 1 < njc:
                plsc.subcore_barrier()

    return pl.kernel(
        body,
        out_type=jax.ShapeDtypeStruct((n_pad, ncols), jnp.float32),
        mesh=mesh,
        scratch_types=(
            [pltpu.VMEM((nke, _EC), jnp.int32)] * 2
            + [pltpu.VMEM((_EC, cw), jnp.float32)] * 4
            + [pltpu.VMEM_SHARED((n_pad, cw), jnp.float32)]
            + [pltpu.SemaphoreType.DMA] * 9
        ),
    )


def kernel(h, h_edge, params, edge_index):
    n, d = h.shape
    e = h_edge.shape[0]
    num_layers = params["phis_W1"].shape[0]
    n_pad = ((n + _BN - 1) // _BN) * _BN
    if n_pad % (_NS * _GC) != 0:
        n_pad = ((n + _NS * _GC - 1) // (_NS * _GC)) * (_NS * _GC)
    e_pad = ((e + _NW * _GC - 1) // (_NW * _GC)) * (_NW * _GC)

    trash = n_pad - 1
    h_p = jnp.pad(h, ((0, n_pad - n), (0, 0)))
    he_p = jnp.pad(h_edge, ((0, e_pad - e), (0, 0)))
    sidx = jnp.pad(edge_index[0], (0, e_pad - e), constant_values=trash)
    oidx = jnp.pad(edge_index[1], (0, e_pad - e), constant_values=trash)
    zeros_blk = jnp.zeros((_GC, _SCW), jnp.float32)
    ones_e = jnp.ones((e_pad, _SCW), jnp.float32)

    gather = _make_gather(n_pad, e_pad)
    scatter = _make_scatter(n_pad, e_pad, _D, _D // (2 * _SCW), False)
    count_k = _make_scatter(n_pad, e_pad, _SCW, 1, True)

    sidx2 = sidx.reshape(e_pad // _EC, _EC)
    oidx2 = oidx.reshape(e_pad // _EC, _EC)

    counts = count_k(ones_e, ones_e, sidx2, oidx2, zeros_blk)[:, :16]

    def wT(x):
        return jnp.swapaxes(x, 0, 1).astype(jnp.bfloat16)

    def row(x):
        return x.reshape(1, -1)

    h_sum = jnp.zeros((n_pad, d), jnp.float32)
    he_sum = jnp.zeros((e_pad, d), jnp.float32)

    for i in range(num_layers):
        ps_all, po_all = _node_mlp(
            h_p,
            wT(params["phis_W1"][i]), row(params["phis_b1"][i]),
            wT(params["phis_W2"][i]), row(params["phis_b2"][i]),
            wT(params["phio_W1"][i]), row(params["phio_b1"][i]),
            wT(params["phio_W2"][i]), row(params["phio_b2"][i]))
        pp = _edge_mlp(
            he_p,
            wT(params["phip_W1"][i]), row(params["phip_b1"][i]),
            wT(params["phip_W2"][i]), row(params["phip_b2"][i]))
        psg, pog = gather(ps_all, po_all, sidx, oidx)
        ms, mo, he_p, he_sum = _edge_mix(
            psg, pog, pp, he_p, he_sum,
            wT(params["edge_gru_Wih"][i]), wT(params["edge_gru_Whh"][i]),
            row(params["edge_gru_bih"][i]), row(params["edge_gru_bhh"][i]),
            row(params["ln_g"][i]), row(params["ln_b"][i]))
        mpool = scatter(ms, mo, sidx2, oidx2, zeros_blk)
        h_p, h_sum = _node_upd(
            mpool, counts, h_p, h_sum,
            wT(params["node_gru_Wih"][i]), wT(params["node_gru_Whh"][i]),
            row(params["node_gru_bih"][i]), row(params["node_gru_bhh"][i]))

    g = row(params["final_ln_g"])
    b = row(params["final_ln_b"])
    h_final = _final_ln(h_sum, g, b)[:n]
    he_final = _final_ln(he_sum, g, b)[:e]
    return (h_final, he_final)


# speedup vs baseline: 2.1750x; 1.0145x over previous
import functools

import jax
import jax.numpy as jnp
from jax import lax
from jax.experimental import pallas as pl
from jax.experimental.pallas import tpu as pltpu
from jax.experimental.pallas import tpu_sc as plsc

_D = 512
_NC = 2
_NS = 16
_NW = _NC * _NS
_BN = 256
_BE = 256
_GC = 128
_SCW = 128
_EC = 32
_NB = 3


def _bdot(x, w):
    return jnp.dot(x.astype(jnp.bfloat16), w,
                   preferred_element_type=jnp.float32)


def _pack2(u, v):
    ui = lax.bitcast_convert_type(u.astype(jnp.bfloat16).astype(jnp.float32),
                                  jnp.uint32)
    vi = lax.bitcast_convert_type(v.astype(jnp.bfloat16).astype(jnp.float32),
                                  jnp.uint32)
    w = (ui >> 16) | (vi & jnp.uint32(0xFFFF0000))
    return lax.bitcast_convert_type(w, jnp.float32)


def _unpack2(w):
    wi = lax.bitcast_convert_type(w, jnp.uint32)
    u = lax.bitcast_convert_type(wi << 16, jnp.float32)
    v = lax.bitcast_convert_type(wi & jnp.uint32(0xFFFF0000), jnp.float32)
    return u, v


def _unpack_cat(w):
    u, v = _unpack2(w)
    return jnp.concatenate([u, v], axis=1)


def _ln(x, g, b, eps=1e-5):
    mu = jnp.mean(x, axis=-1, keepdims=True)
    xc = x - mu
    var = jnp.mean(xc * xc, axis=-1, keepdims=True)
    return xc * jax.lax.rsqrt(var + eps) * g + b


def _node_mlp_body(h_ref, w1s, b1s, w2s, b2s, w1o, b1o, w2o, b2o, ps_o, po_o):
    hb = h_ref[...]
    hw = _D // 2
    t = jnp.maximum(_bdot(hb, w1s[...]) + b1s[...], 0.0)
    z = _bdot(t, w2s[...]) + b2s[...]
    ps_o[...] = _pack2(z[:, :hw], z[:, hw:])
    t = jnp.maximum(_bdot(hb, w1o[...]) + b1o[...], 0.0)
    z = _bdot(t, w2o[...]) + b2o[...]
    po_o[...] = _pack2(z[:, :hw], z[:, hw:])


def _node_mlp(h_p, w1s, b1s, w2s, b2s, w1o, b1o, w2o, b2o):
    n_pad = h_p.shape[0]
    blk = pl.BlockSpec((_BN, _D), lambda i: (i, 0))
    pblk = pl.BlockSpec((_BN, _D // 2), lambda i: (i, 0))
    wsp = pl.BlockSpec((_D, _D), lambda i: (0, 0))
    bsp = pl.BlockSpec((1, _D), lambda i: (0, 0))
    return pl.pallas_call(
        _node_mlp_body,
        grid=(n_pad // _BN,),
        in_specs=[blk, wsp, bsp, wsp, bsp, wsp, bsp, wsp, bsp],
        out_specs=(pblk, pblk),
        out_shape=(jax.ShapeDtypeStruct((n_pad, _D // 2), jnp.float32),) * 2,
    )(h_p, w1s, b1s, w2s, b2s, w1o, b1o, w2o, b2o)


def _edge_mlp_body(x_ref, w1, b1, w2, b2, out_o):
    t = jnp.maximum(_bdot(x_ref[...], w1[...]) + b1[...], 0.0)
    z = _bdot(t, w2[...]) + b2[...]
    out_o[...] = _pack2(z[:, :_D // 2], z[:, _D // 2:])


def _edge_mlp(x, w1, b1, w2, b2):
    e_pad = x.shape[0]
    blk = pl.BlockSpec((_BE, _D), lambda i: (i, 0))
    pblk = pl.BlockSpec((_BE, _D // 2), lambda i: (i, 0))
    wsp = pl.BlockSpec((_D, _D), lambda i: (0, 0))
    bsp = pl.BlockSpec((1, _D), lambda i: (0, 0))
    return pl.pallas_call(
        _edge_mlp_body,
        grid=(e_pad // _BE,),
        in_specs=[blk, wsp, bsp, wsp, bsp],
        out_specs=pblk,
        out_shape=jax.ShapeDtypeStruct((e_pad, _D // 2), jnp.float32),
    )(x, w1, b1, w2, b2)


def _gru(x, h, wih, whh, bih, bhh):
    gi = _bdot(x, wih) + bih
    gh = _bdot(h, whh) + bhh
    r = jax.nn.sigmoid(gi[:, :_D] + gh[:, :_D])
    z = jax.nn.sigmoid(gi[:, _D:2 * _D] + gh[:, _D:2 * _D])
    n = jnp.tanh(gi[:, 2 * _D:] + r * gh[:, 2 * _D:])
    return (1.0 - z) * n + z * h


def _edge_mix_body(psg, pog, ppb, he, hes, wih, whh, bih, bhh, g, b,
                   ms_o, mo_o, he_o, hes_o):
    a = _unpack_cat(psg[...])
    o = _unpack_cat(pog[...])
    p = _unpack_cat(ppb[...])
    gv = g[...]
    bv = b[...]
    ms_o[...] = _ln(o + p, gv, bv)
    mo_o[...] = _ln(a + p, gv, bv)
    mp = _ln(a + o, gv, bv)
    hn = _gru(mp, he[...], wih[...], whh[...], bih[...], bhh[...])
    he_o[...] = hn
    hes_o[...] = hes[...] + hn


def _edge_mix(psg, pog, pp, he, hes, wih, whh, bih, bhh, g, b):
    e_pad = psg.shape[0]
    blk = pl.BlockSpec((_BE, _D), lambda i: (i, 0))
    pblk = pl.BlockSpec((_BE, _D // 2), lambda i: (i, 0))
    wsp = pl.BlockSpec((_D, 3 * _D), lambda i: (0, 0))
    b3 = pl.BlockSpec((1, 3 * _D), lambda i: (0, 0))
    b1 = pl.BlockSpec((1, _D), lambda i: (0, 0))
    return pl.pallas_call(
        _edge_mix_body,
        grid=(e_pad // _BE,),
        in_specs=[pblk, pblk, pblk, blk, blk, wsp, wsp, b3, b3, b1, b1],
        out_specs=(blk, blk, blk, blk),
        out_shape=(jax.ShapeDtypeStruct((e_pad, _D), jnp.float32),) * 4,
    )(psg, pog, pp, he, hes, wih, whh, bih, bhh, g, b)


def _node_upd_body(mpool, cnt, h, hs, wih, whh, bih, bhh, h_o, hs_o):
    c = jnp.maximum(cnt[...][:, :1], 1.0)
    mn = mpool[...] / c
    hn = _gru(mn, h[...], wih[...], whh[...], bih[...], bhh[...])
    h_o[...] = hn
    hs_o[...] = hs[...] + hn


def _node_upd(mpool, cnt, h, hs, wih, whh, bih, bhh):
    n_pad = mpool.shape[0]
    blk = pl.BlockSpec((_BN, _D), lambda i: (i, 0))
    cblk = pl.BlockSpec((_BN, 16), lambda i: (i, 0))
    wsp = pl.BlockSpec((_D, 3 * _D), lambda i: (0, 0))
    b3 = pl.BlockSpec((1, 3 * _D), lambda i: (0, 0))
    return pl.pallas_call(
        _node_upd_body,
        grid=(n_pad // _BN,),
        in_specs=[blk, cblk, blk, blk, wsp, wsp, b3, b3],
        out_specs=(blk, blk),
        out_shape=(jax.ShapeDtypeStruct((n_pad, _D), jnp.float32),) * 2,
    )(mpool, cnt, h, hs, wih, whh, bih, bhh)


def _final_ln_body(x, g, b, o):
    o[...] = _ln(x[...], g[...], b[...])


def _final_ln(x, g, b):
    rows = x.shape[0]
    blk = pl.BlockSpec((_BE, _D), lambda i: (i, 0))
    bsp = pl.BlockSpec((1, _D), lambda i: (0, 0))
    return pl.pallas_call(
        _final_ln_body,
        grid=(rows // _BE,),
        in_specs=[blk, bsp, bsp],
        out_specs=blk,
        out_shape=jax.ShapeDtypeStruct((rows, _D), jnp.float32),
    )(x, g, b)


def _make_gather(n_pad, e_pad):
    dw = _D // 2
    epw = e_pad // _NW
    gc = 64
    nk = epw // gc
    assert epw % gc == 0
    mesh = plsc.VectorSubcoreMesh(core_axis_name="c", subcore_axis_name="s",
                                  num_cores=_NC, num_subcores=_NS)

    def body(ps_hbm, po_hbm, sidx_hbm, oidx_hbm, psg_out, pog_out,
             idx_v, r00, r01, r02, r10, r11, r12,
             gs00, gs01, gs02, gs10, gs11, gs12,
             os00, os01, os02, os10, os11, os12):
        rows = ((r00, r01, r02), (r10, r11, r12))
        gsem = ((gs00, gs01, gs02), (gs10, gs11, gs12))
        osem = ((os00, os01, os02), (os10, os11, os12))
        c = lax.axis_index("c")
        s = lax.axis_index("s")
        wid = s * _NC + c
        base = wid * epw
        tabs = ((ps_hbm, sidx_hbm, psg_out), (po_hbm, oidx_hbm, pog_out))
        pltpu.sync_copy(tabs[0][1].at[pl.ds(base, epw)], idx_v.at[0])
        pltpu.sync_copy(tabs[1][1].at[pl.ds(base, epw)], idx_v.at[1])

        def gstart(t, k):
            b = k % _NB
            return pltpu.async_copy(
                tabs[t][0].at[idx_v.at[t, pl.ds(k * gc, gc)]],
                rows[t][b], gsem[t][b])

        def ostart(t, k):
            b = k % _NB
            return pltpu.async_copy(
                rows[t][b], tabs[t][2].at[pl.ds(base + k * gc, gc), :],
                osem[t][b])

        gh = [[None] * nk, [None] * nk]
        oh = [[None] * nk, [None] * nk]
        for i in range(min(_NB - 1, nk)):
            gh[0][i] = gstart(0, i)
            gh[1][i] = gstart(1, i)
        for k in range(nk):
            for t in (0, 1):
                m = k + _NB - 1
                if m < nk:
                    if m - _NB >= 0:
                        oh[t][m - _NB].wait()
                    gh[t][m] = gstart(t, m)
                gh[t][k].wait()
                oh[t][k] = ostart(t, k)
        for t in (0, 1):
            for k in range(max(0, nk - _NB), nk):
                oh[t][k].wait()

    return pl.kernel(
        body,
        out_type=(jax.ShapeDtypeStruct((e_pad, dw), jnp.float32),) * 2,
        mesh=mesh,
        scratch_types=(
            [pltpu.VMEM((2, epw), jnp.int32)]
            + [pltpu.VMEM((gc, dw), jnp.float32)] * 6
            + [pltpu.SemaphoreType.DMA] * 12
        ),
    )


def _make_scatter(n_pad, e_pad, ncols, njc, writeback_core0_only):
    cw = min(_SCW, ncols)
    ept = e_pad // _NS
    npt = n_pad // _NS
    nke = ept // _EC
    nkn = npt // _EC
    assert ept % _EC == 0 and npt % _EC == 0
    mesh = plsc.VectorSubcoreMesh(core_axis_name="c", subcore_axis_name="s",
                                  num_cores=_NC, num_subcores=_NS)

    def body(ms_hbm, mo_hbm, sidx_hbm, oidx_hbm, zeros_hbm, out_hbm,
             idx_s, idx_o, v00, v01, v02, v10, v11, v12, acc_sp,
             vs00, vs01, vs02, vs10, vs11, vs12,
             as00, as01, as02, as10, as11, as12, osem):
        val = ((v00, v01, v02), (v10, v11, v12))
        vsem = ((vs00, vs01, vs02), (vs10, vs11, vs12))
        asem = ((as00, as01, as02), (as10, as11, as12))
        c = lax.axis_index("c")
        s = lax.axis_index("s")
        pltpu.sync_copy(sidx_hbm.at[pl.ds(s * nke, nke), :], idx_s)
        pltpu.sync_copy(oidx_hbm.at[pl.ds(s * nke, nke), :], idx_o)
        for j in range(njc):
            if writeback_core0_only:
                col = j * cw
            else:
                col = c * cw + j * (2 * cw)
            pltpu.sync_copy(zeros_hbm.at[pl.ds(0, _EC), pl.ds(0, cw)], v00)
            zh = [pltpu.async_copy(v00, acc_sp.at[pl.ds(s * npt + k * _EC,
                                                        _EC), :], vs00)
                  for k in range(nkn)]
            for h in zh:
                h.wait()
            plsc.subcore_barrier()

            arrs = ((ms_hbm, idx_s), (mo_hbm, idx_o))

            def vstart(t, k):
                b = k % _NB
                r0 = s * ept + k * _EC
                return pltpu.async_copy(
                    arrs[t][0].at[pl.ds(r0, _EC), pl.ds(col, cw)],
                    val[t][b], vsem[t][b])

            def astart(t, k):
                b = k % _NB
                return pltpu.async_copy(val[t][b],
                                        acc_sp.at[arrs[t][1].at[k]],
                                        asem[t][b], add=True)

            vh = [[None] * nke, [None] * nke]
            ah = [[None] * nke, [None] * nke]
            for i in range(min(_NB - 1, nke)):
                vh[0][i] = vstart(0, i)
                vh[1][i] = vstart(1, i)
            for k in range(nke):
                for t in (0, 1):
                    m = k + _NB - 1
                    if m < nke:
                        if m - _NB >= 0:
                            ah[t][m - _NB].wait()
                        vh[t][m] = vstart(t, m)
                    vh[t][k].wait()
                    ah[t][k] = astart(t, k)
            for t in (0, 1):
                for k in range(max(0, nke - _NB), nke):
                    ah[t][k].wait()
            plsc.subcore_barrier()

            def writeback():
                wh = [None] * nkn
                for k in range(nkn):
                    b = k % 2
                    if k >= 2:
                        wh[k - 2].wait()
                    r0 = s * npt + k * _EC
                    pltpu.sync_copy(acc_sp.at[pl.ds(r0, _EC), :], val[0][b])
                    wh[k] = pltpu.async_copy(
                        val[0][b], out_hbm.at[pl.ds(r0, _EC), pl.ds(col, cw)],
                        osem)
                for k in range(max(0, nkn - 2), nkn):
                    wh[k].wait()

            if writeback_core0_only:
                pl.when(c == 0)(writeback)
            else:
                writeback()
            if j + 1 < njc:
                plsc.subcore_barrier()

    return pl.kernel(
        body,
        out_type=jax.ShapeDtypeStruct((n_pad, ncols), jnp.float32),
        mesh=mesh,
        scratch_types=(
            [pltpu.VMEM((nke, _EC), jnp.int32)] * 2
            + [pltpu.VMEM((_EC, cw), jnp.float32)] * 6
            + [pltpu.VMEM_SHARED((n_pad, cw), jnp.float32)]
            + [pltpu.SemaphoreType.DMA] * 13
        ),
    )


def kernel(h, h_edge, params, edge_index):
    n, d = h.shape
    e = h_edge.shape[0]
    num_layers = params["phis_W1"].shape[0]
    n_pad = ((n + _BN - 1) // _BN) * _BN
    if n_pad % (_NS * _GC) != 0:
        n_pad = ((n + _NS * _GC - 1) // (_NS * _GC)) * (_NS * _GC)
    e_pad = ((e + _NW * _GC - 1) // (_NW * _GC)) * (_NW * _GC)

    trash = n_pad - 1
    h_p = jnp.pad(h, ((0, n_pad - n), (0, 0)))
    he_p = jnp.pad(h_edge, ((0, e_pad - e), (0, 0)))
    sidx = jnp.pad(edge_index[0], (0, e_pad - e), constant_values=trash)
    oidx = jnp.pad(edge_index[1], (0, e_pad - e), constant_values=trash)
    zeros_blk = jnp.zeros((_GC, _SCW), jnp.float32)
    ones_e = jnp.ones((e_pad, _SCW), jnp.float32)

    gather = _make_gather(n_pad, e_pad)
    scatter = _make_scatter(n_pad, e_pad, _D, _D // (2 * _SCW), False)
    count_k = _make_scatter(n_pad, e_pad, _SCW, 1, True)

    sidx2 = sidx.reshape(e_pad // _EC, _EC)
    oidx2 = oidx.reshape(e_pad // _EC, _EC)

    counts = count_k(ones_e, ones_e, sidx2, oidx2, zeros_blk)[:, :16]

    def wT(x):
        return jnp.swapaxes(x, 0, 1).astype(jnp.bfloat16)

    def row(x):
        return x.reshape(1, -1)

    h_sum = jnp.zeros((n_pad, d), jnp.float32)
    he_sum = jnp.zeros((e_pad, d), jnp.float32)

    for i in range(num_layers):
        ps_all, po_all = _node_mlp(
            h_p,
            wT(params["phis_W1"][i]), row(params["phis_b1"][i]),
            wT(params["phis_W2"][i]), row(params["phis_b2"][i]),
            wT(params["phio_W1"][i]), row(params["phio_b1"][i]),
            wT(params["phio_W2"][i]), row(params["phio_b2"][i]))
        pp = _edge_mlp(
            he_p,
            wT(params["phip_W1"][i]), row(params["phip_b1"][i]),
            wT(params["phip_W2"][i]), row(params["phip_b2"][i]))
        psg, pog = gather(ps_all, po_all, sidx, oidx)
        ms, mo, he_p, he_sum = _edge_mix(
            psg, pog, pp, he_p, he_sum,
            wT(params["edge_gru_Wih"][i]), wT(params["edge_gru_Whh"][i]),
            row(params["edge_gru_bih"][i]), row(params["edge_gru_bhh"][i]),
            row(params["ln_g"][i]), row(params["ln_b"][i]))
        mpool = scatter(ms, mo, sidx2, oidx2, zeros_blk)
        h_p, h_sum = _node_upd(
            mpool, counts, h_p, h_sum,
            wT(params["node_gru_Wih"][i]), wT(params["node_gru_Whh"][i]),
            row(params["node_gru_bih"][i]), row(params["node_gru_bhh"][i]))

    g = row(params["final_ln_g"])
    b = row(params["final_ln_b"])
    h_final = _final_ln(h_sum, g, b)[:n]
    he_final = _final_ln(he_sum, g, b)[:e]
    return (h_final, he_final)


# speedup vs baseline: 2.2073x; 1.0148x over previous
import jax
import jax.numpy as jnp
from jax import lax
from jax.experimental import pallas as pl
from jax.experimental.pallas import tpu as pltpu
from jax.experimental.pallas import tpu_sc as plsc

_D = 512
_NC = 2
_NS = 16
_NW = _NC * _NS
_BN = 256
_BE = 256
_GC = 128
_SCW = 128
_EC = 32
_NB = 3


def _bdot(x, w):
    return jnp.dot(x.astype(jnp.bfloat16), w,
                   preferred_element_type=jnp.float32)


def _pack2(u, v):
    ui = lax.bitcast_convert_type(u.astype(jnp.bfloat16).astype(jnp.float32),
                                  jnp.uint32)
    vi = lax.bitcast_convert_type(v.astype(jnp.bfloat16).astype(jnp.float32),
                                  jnp.uint32)
    w = (ui >> 16) | (vi & jnp.uint32(0xFFFF0000))
    return lax.bitcast_convert_type(w, jnp.float32)


def _unpack2(w):
    wi = lax.bitcast_convert_type(w, jnp.uint32)
    u = lax.bitcast_convert_type(wi << 16, jnp.float32)
    v = lax.bitcast_convert_type(wi & jnp.uint32(0xFFFF0000), jnp.float32)
    return u, v


def _unpack_cat(w):
    u, v = _unpack2(w)
    return jnp.concatenate([u, v], axis=1)


def _ln(x, g, b, eps=1e-5):
    mu = jnp.mean(x, axis=-1, keepdims=True)
    xc = x - mu
    var = jnp.mean(xc * xc, axis=-1, keepdims=True)
    return xc * jax.lax.rsqrt(var + eps) * g + b


def _node_mlp_body(h_ref, w1s, b1s, w2s, b2s, w1o, b1o, w2o, b2o, ps_o, po_o):
    hb = h_ref[...]
    hw = _D // 2
    t = jnp.maximum(_bdot(hb, w1s[...]) + b1s[...], 0.0)
    z = _bdot(t, w2s[...]) + b2s[...]
    ps_o[...] = _pack2(z[:, :hw], z[:, hw:])
    t = jnp.maximum(_bdot(hb, w1o[...]) + b1o[...], 0.0)
    z = _bdot(t, w2o[...]) + b2o[...]
    po_o[...] = _pack2(z[:, :hw], z[:, hw:])


def _node_mlp(h_p, w1s, b1s, w2s, b2s, w1o, b1o, w2o, b2o):
    n_pad = h_p.shape[0]
    blk = pl.BlockSpec((_BN, _D), lambda i: (i, 0))
    pblk = pl.BlockSpec((_BN, _D // 2), lambda i: (i, 0))
    wsp = pl.BlockSpec((_D, _D), lambda i: (0, 0))
    bsp = pl.BlockSpec((1, _D), lambda i: (0, 0))
    return pl.pallas_call(
        _node_mlp_body,
        grid=(n_pad // _BN,),
        in_specs=[blk, wsp, bsp, wsp, bsp, wsp, bsp, wsp, bsp],
        out_specs=(pblk, pblk),
        out_shape=(jax.ShapeDtypeStruct((n_pad, _D // 2), jnp.float32),) * 2,
    )(h_p, w1s, b1s, w2s, b2s, w1o, b1o, w2o, b2o)


def _edge_mlp_body(x_ref, w1, b1, w2, b2, out_o):
    t = jnp.maximum(_bdot(x_ref[...], w1[...]) + b1[...], 0.0)
    z = _bdot(t, w2[...]) + b2[...]
    out_o[...] = _pack2(z[:, :_D // 2], z[:, _D // 2:])


def _edge_mlp(x, w1, b1, w2, b2):
    e_pad = x.shape[0]
    blk = pl.BlockSpec((_BE, _D), lambda i: (i, 0))
    pblk = pl.BlockSpec((_BE, _D // 2), lambda i: (i, 0))
    wsp = pl.BlockSpec((_D, _D), lambda i: (0, 0))
    bsp = pl.BlockSpec((1, _D), lambda i: (0, 0))
    return pl.pallas_call(
        _edge_mlp_body,
        grid=(e_pad // _BE,),
        in_specs=[blk, wsp, bsp, wsp, bsp],
        out_specs=pblk,
        out_shape=jax.ShapeDtypeStruct((e_pad, _D // 2), jnp.float32),
    )(x, w1, b1, w2, b2)


def _gru(x, h, wih, whh, bih, bhh):
    gi = _bdot(x, wih) + bih
    gh = _bdot(h, whh) + bhh
    r = jax.nn.sigmoid(gi[:, :_D] + gh[:, :_D])
    z = jax.nn.sigmoid(gi[:, _D:2 * _D] + gh[:, _D:2 * _D])
    n = jnp.tanh(gi[:, 2 * _D:] + r * gh[:, 2 * _D:])
    return (1.0 - z) * n + z * h


def _edge_mix_body(psg, pog, ppb, he, hes, wih, whh, bih, bhh, g, b,
                   ms_o, mo_o, he_o, hes_o):
    a = _unpack_cat(psg[...])
    o = _unpack_cat(pog[...])
    p = _unpack_cat(ppb[...])
    gv = g[...]
    bv = b[...]
    ms_o[...] = _ln(o + p, gv, bv)
    mo_o[...] = _ln(a + p, gv, bv)
    mp = _ln(a + o, gv, bv)
    hn = _gru(mp, he[...], wih[...], whh[...], bih[...], bhh[...])
    he_o[...] = hn
    hes_o[...] = hes[...] + hn


def _edge_mix(psg, pog, pp, he, hes, wih, whh, bih, bhh, g, b):
    e_pad = psg.shape[0]
    blk = pl.BlockSpec((_BE, _D), lambda i: (i, 0))
    pblk = pl.BlockSpec((_BE, _D // 2), lambda i: (i, 0))
    wsp = pl.BlockSpec((_D, 3 * _D), lambda i: (0, 0))
    b3 = pl.BlockSpec((1, 3 * _D), lambda i: (0, 0))
    b1 = pl.BlockSpec((1, _D), lambda i: (0, 0))
    return pl.pallas_call(
        _edge_mix_body,
        grid=(e_pad // _BE,),
        in_specs=[pblk, pblk, pblk, blk, blk, wsp, wsp, b3, b3, b1, b1],
        out_specs=(blk, blk, blk, blk),
        out_shape=(jax.ShapeDtypeStruct((e_pad, _D), jnp.float32),) * 4,
    )(psg, pog, pp, he, hes, wih, whh, bih, bhh, g, b)


def _node_upd_body(mpool, cnt, h, hs, wih, whh, bih, bhh, h_o, hs_o):
    c = jnp.maximum(cnt[...][:, :1], 1.0)
    mn = mpool[...] / c
    hn = _gru(mn, h[...], wih[...], whh[...], bih[...], bhh[...])
    h_o[...] = hn
    hs_o[...] = hs[...] + hn


def _node_upd(mpool, cnt, h, hs, wih, whh, bih, bhh):
    n_pad = mpool.shape[0]
    blk = pl.BlockSpec((_BN, _D), lambda i: (i, 0))
    cblk = pl.BlockSpec((_BN, 16), lambda i: (i, 0))
    wsp = pl.BlockSpec((_D, 3 * _D), lambda i: (0, 0))
    b3 = pl.BlockSpec((1, 3 * _D), lambda i: (0, 0))
    return pl.pallas_call(
        _node_upd_body,
        grid=(n_pad // _BN,),
        in_specs=[blk, cblk, blk, blk, wsp, wsp, b3, b3],
        out_specs=(blk, blk),
        out_shape=(jax.ShapeDtypeStruct((n_pad, _D), jnp.float32),) * 2,
    )(mpool, cnt, h, hs, wih, whh, bih, bhh)


def _final_ln_body(x, g, b, o):
    o[...] = _ln(x[...], g[...], b[...])


def _final_ln(x, g, b):
    rows = x.shape[0]
    blk = pl.BlockSpec((_BE, _D), lambda i: (i, 0))
    bsp = pl.BlockSpec((1, _D), lambda i: (0, 0))
    return pl.pallas_call(
        _final_ln_body,
        grid=(rows // _BE,),
        in_specs=[blk, bsp, bsp],
        out_specs=blk,
        out_shape=jax.ShapeDtypeStruct((rows, _D), jnp.float32),
    )(x, g, b)


def _make_gather(n_pad, e_pad):
    dw = _D // 2
    epw = e_pad // _NW
    gc = 64
    nk = epw // gc
    assert epw % gc == 0
    mesh = plsc.VectorSubcoreMesh(core_axis_name="c", subcore_axis_name="s",
                                  num_cores=_NC, num_subcores=_NS)

    def body(ps_hbm, po_hbm, sidx_hbm, oidx_hbm, psg_out, pog_out,
             idx_v, r00, r01, r02, r10, r11, r12,
             gs00, gs01, gs02, gs10, gs11, gs12,
             os00, os01, os02, os10, os11, os12):
        rows = ((r00, r01, r02), (r10, r11, r12))
        gsem = ((gs00, gs01, gs02), (gs10, gs11, gs12))
        osem = ((os00, os01, os02), (os10, os11, os12))
        c = lax.axis_index("c")
        s = lax.axis_index("s")
        wid = s * _NC + c
        base = wid * epw
        tabs = ((ps_hbm, sidx_hbm, psg_out), (po_hbm, oidx_hbm, pog_out))
        pltpu.sync_copy(tabs[0][1].at[pl.ds(base, epw)], idx_v.at[0])
        pltpu.sync_copy(tabs[1][1].at[pl.ds(base, epw)], idx_v.at[1])

        def gstart(t, k):
            b = k % _NB
            return pltpu.async_copy(
                tabs[t][0].at[idx_v.at[t, pl.ds(k * gc, gc)]],
                rows[t][b], gsem[t][b])

        def ostart(t, k):
            b = k % _NB
            return pltpu.async_copy(
                rows[t][b], tabs[t][2].at[pl.ds(base + k * gc, gc), :],
                osem[t][b])

        gh = [[None] * nk, [None] * nk]
        oh = [[None] * nk, [None] * nk]
        for i in range(min(_NB - 1, nk)):
            gh[0][i] = gstart(0, i)
            gh[1][i] = gstart(1, i)
        for k in range(nk):
            for t in (0, 1):
                m = k + _NB - 1
                if m < nk:
                    if m - _NB >= 0:
                        oh[t][m - _NB].wait()
                    gh[t][m] = gstart(t, m)
                gh[t][k].wait()
                oh[t][k] = ostart(t, k)
        for t in (0, 1):
            for k in range(max(0, nk - _NB), nk):
                oh[t][k].wait()

    return pl.kernel(
        body,
        out_type=(jax.ShapeDtypeStruct((e_pad, dw), jnp.float32),) * 2,
        mesh=mesh,
        scratch_types=(
            [pltpu.VMEM((2, epw), jnp.int32)]
            + [pltpu.VMEM((gc, dw), jnp.float32)] * 6
            + [pltpu.SemaphoreType.DMA] * 12
        ),
    )


def _make_scatter(n_pad, e_pad, ncols, njc, writeback_core0_only):
    cw = min(_SCW, ncols)
    ept = e_pad // _NS
    npt = n_pad // _NS
    nke = ept // _EC
    nkn = npt // _EC
    assert ept % _EC == 0 and npt % _EC == 0
    mesh = plsc.VectorSubcoreMesh(core_axis_name="c", subcore_axis_name="s",
                                  num_cores=_NC, num_subcores=_NS)

    def body(ms_hbm, mo_hbm, sidx_hbm, oidx_hbm, zeros_hbm, out_hbm,
             idx_s, idx_o, v00, v01, v02, v10, v11, v12, acc_sp,
             vs00, vs01, vs02, vs10, vs11, vs12,
             as00, as01, as02, as10, as11, as12, osem):
        val = ((v00, v01, v02), (v10, v11, v12))
        vsem = ((vs00, vs01, vs02), (vs10, vs11, vs12))
        asem = ((as00, as01, as02), (as10, as11, as12))
        c = lax.axis_index("c")
        s = lax.axis_index("s")
        pltpu.sync_copy(sidx_hbm.at[pl.ds(s * nke, nke), :], idx_s)
        pltpu.sync_copy(oidx_hbm.at[pl.ds(s * nke, nke), :], idx_o)
        for j in range(njc):
            if writeback_core0_only:
                col = j * cw
            else:
                col = c * cw + j * (2 * cw)
            pltpu.sync_copy(zeros_hbm.at[pl.ds(0, _EC), pl.ds(0, cw)], v00)
            zh = [pltpu.async_copy(v00, acc_sp.at[pl.ds(s * npt + k * _EC,
                                                        _EC), :], vs00)
                  for k in range(nkn)]
            for h in zh:
                h.wait()
            plsc.subcore_barrier()

            arrs = ((ms_hbm, idx_s), (mo_hbm, idx_o))

            def vstart(t, k):
                b = k % _NB
                r0 = s * ept + k * _EC
                return pltpu.async_copy(
                    arrs[t][0].at[pl.ds(r0, _EC), pl.ds(col, cw)],
                    val[t][b], vsem[t][b])

            def astart(t, k):
                b = k % _NB
                return pltpu.async_copy(val[t][b],
                                        acc_sp.at[arrs[t][1].at[k]],
                                        asem[t][b], add=True)

            vh = [[None] * nke, [None] * nke]
            ah = [[None] * nke, [None] * nke]
            for i in range(min(_NB - 1, nke)):
                vh[0][i] = vstart(0, i)
                vh[1][i] = vstart(1, i)
            for k in range(nke):
                for t in (0, 1):
                    m = k + _NB - 1
                    if m < nke:
                        if m - _NB >= 0:
                            ah[t][m - _NB].wait()
                        vh[t][m] = vstart(t, m)
                    vh[t][k].wait()
                    ah[t][k] = astart(t, k)
            for t in (0, 1):
                for k in range(max(0, nke - _NB), nke):
                    ah[t][k].wait()
            plsc.subcore_barrier()

            def writeback():
                wh = [None] * nkn
                for k in range(nkn):
                    b = k % 2
                    if k >= 2:
                        wh[k - 2].wait()
                    r0 = s * npt + k * _EC
                    pltpu.sync_copy(acc_sp.at[pl.ds(r0, _EC), :], val[0][b])
                    wh[k] = pltpu.async_copy(
                        val[0][b], out_hbm.at[pl.ds(r0, _EC), pl.ds(col, cw)],
                        osem)
                for k in range(max(0, nkn - 2), nkn):
                    wh[k].wait()

            if writeback_core0_only:
                pl.when(c == 0)(writeback)
            else:
                writeback()
            if j + 1 < njc:
                plsc.subcore_barrier()

    return pl.kernel(
        body,
        out_type=jax.ShapeDtypeStruct((n_pad, ncols), jnp.float32),
        mesh=mesh,
        scratch_types=(
            [pltpu.VMEM((nke, _EC), jnp.int32)] * 2
            + [pltpu.VMEM((_EC, cw), jnp.float32)] * 6
            + [pltpu.VMEM_SHARED((n_pad, cw), jnp.float32)]
            + [pltpu.SemaphoreType.DMA] * 13
        ),
    )


def kernel(h, h_edge, params, edge_index):
    n, d = h.shape
    e = h_edge.shape[0]
    num_layers = params["phis_W1"].shape[0]
    n_pad = ((n + _BN - 1) // _BN) * _BN
    if n_pad % (_NS * _GC) != 0:
        n_pad = ((n + _NS * _GC - 1) // (_NS * _GC)) * (_NS * _GC)
    e_pad = ((e + _NW * _GC - 1) // (_NW * _GC)) * (_NW * _GC)

    trash = n_pad - 1
    h_p = jnp.pad(h, ((0, n_pad - n), (0, 0)))
    he_p = jnp.pad(h_edge, ((0, e_pad - e), (0, 0)))
    sidx = jnp.pad(edge_index[0], (0, e_pad - e), constant_values=trash)
    oidx = jnp.pad(edge_index[1], (0, e_pad - e), constant_values=trash)
    zeros_blk = jnp.zeros((_GC, _SCW), jnp.float32)
    ones_e = jnp.ones((e_pad, _SCW), jnp.float32)

    gather = _make_gather(n_pad, e_pad)
    scatter = _make_scatter(n_pad, e_pad, _D, _D // (2 * _SCW), False)
    count_k = _make_scatter(n_pad, e_pad, _SCW, 1, True)

    sidx2 = sidx.reshape(e_pad // _EC, _EC)
    oidx2 = oidx.reshape(e_pad // _EC, _EC)

    counts = count_k(ones_e, ones_e, sidx2, oidx2, zeros_blk)[:, :16]

    def wT(x):
        return jnp.swapaxes(x, 0, 1).astype(jnp.bfloat16)

    def row(x):
        return x.reshape(1, -1)

    h_sum = jnp.zeros((n_pad, d), jnp.float32)
    he_sum = jnp.zeros((e_pad, d), jnp.float32)

    for i in range(num_layers):
        ps_all, po_all = _node_mlp(
            h_p,
            wT(params["phis_W1"][i]), row(params["phis_b1"][i]),
            wT(params["phis_W2"][i]), row(params["phis_b2"][i]),
            wT(params["phio_W1"][i]), row(params["phio_b1"][i]),
            wT(params["phio_W2"][i]), row(params["phio_b2"][i]))
        pp = _edge_mlp(
            he_p,
            wT(params["phip_W1"][i]), row(params["phip_b1"][i]),
            wT(params["phip_W2"][i]), row(params["phip_b2"][i]))
        psg, pog = gather(ps_all, po_all, sidx, oidx)
        ms, mo, he_p, he_sum = _edge_mix(
            psg, pog, pp, he_p, he_sum,
            wT(params["edge_gru_Wih"][i]), wT(params["edge_gru_Whh"][i]),
            row(params["edge_gru_bih"][i]), row(params["edge_gru_bhh"][i]),
            row(params["ln_g"][i]), row(params["ln_b"][i]))
        mpool = scatter(ms, mo, sidx2, oidx2, zeros_blk)
        h_p, h_sum = _node_upd(
            mpool, counts, h_p, h_sum,
            wT(params["node_gru_Wih"][i]), wT(params["node_gru_Whh"][i]),
            row(params["node_gru_bih"][i]), row(params["node_gru_bhh"][i]))

    g = row(params["final_ln_g"])
    b = row(params["final_ln_b"])
    h_final = _final_ln(h_sum, g, b)[:n]
    he_final = _final_ln(he_sum, g, b)[:e]
    return (h_final, he_final)
